# baseline (speedup 1.0000x reference)
import jax
import jax.numpy as jnp
from jax import lax
from jax.experimental import pallas as pl
from jax.experimental.pallas import tpu as pltpu

NUM_CLS = 151
C_PAD = 152
G_ROWS = 19
EMBED_DIM = 200
HIDDEN = 512
N_PER = 256
N_IMG = 4
NMS_THRESH = 0.5


def _decode_kernel(featsT_ref, wT_ref, b_ref, geom_ref, tableT_ref,
                   distsT_ref, labels_ref, embedT_ref, pgrid_ref):
    row_iota = lax.broadcasted_iota(jnp.int32, (C_PAD, N_PER), 0)
    lane3 = lax.broadcasted_iota(jnp.int32, (1, 1, N_PER), 2)
    lane1 = lax.broadcasted_iota(jnp.int32, (1, N_PER), 1)
    sub8 = lax.broadcasted_iota(jnp.int32, (8, N_PER), 0)
    iota_col = lax.broadcasted_iota(jnp.int32, (N_PER, 1), 0)
    rowc = lax.broadcasted_iota(jnp.int32, (C_PAD, 1), 0)
    big = jnp.int32(1 << 30)

    for i in range(N_IMG):
        dT = jnp.dot(wT_ref[...], featsT_ref[i],
                     preferred_element_type=jnp.float32) + b_ref[...]
        dT = jnp.where(row_iota >= NUM_CLS, -1e30, dT)
        distsT_ref[i] = dT
        m = jnp.max(dT, axis=0, keepdims=True)
        e = jnp.exp(dT - m)
        p = e / jnp.sum(e, axis=0, keepdims=True)
        p = jnp.where(row_iota == 0, -1.0, p)
        p = jnp.where(row_iota >= NUM_CLS, -1e9, p)
        pgrid_ref[i] = p.reshape(G_ROWS, 8, N_PER)

    def body(_, carry):
        out = []
        for i in range(N_IMG):
            dead, labels = carry[i]
            pall = pgrid_ref[i]
            peff = jnp.where(dead != 0, -1.0, pall)
            m1 = jnp.max(jnp.max(peff, axis=0, keepdims=True),
                         axis=1, keepdims=True)
            mx = jnp.max(m1, axis=2, keepdims=True)
            box_v = jnp.min(jnp.where(m1 == mx, lane3, big),
                            axis=2, keepdims=True)
            onehot = (iota_col == box_v.reshape(1, 1)).astype(jnp.float32)
            colvec = jnp.dot(pall.reshape(C_PAD, N_PER), onehot,
                             preferred_element_type=jnp.float32,
                             precision=lax.Precision.HIGHEST)
            cls = jnp.min(jnp.where(colvec == mx.reshape(1, 1), rowc, big))

            g = geom_ref[i, cls]
            pc = jnp.dot(g, onehot,
                         preferred_element_type=jnp.float32,
                         precision=lax.Precision.HIGHEST)
            x1 = g[0:1]
            y1 = g[1:2]
            x2 = g[2:3]
            y2 = g[3:4]
            px1 = pc[0:1]
            py1 = pc[1:2]
            px2 = pc[2:3]
            py2 = pc[3:4]

            iw = jnp.maximum(jnp.minimum(px2, x2) - jnp.maximum(px1, x1) + 1.0, 0.0)
            ih = jnp.maximum(jnp.minimum(py2, y2) - jnp.maximum(py1, y1) + 1.0, 0.0)
            inter = iw * ih
            areas = (x2 - x1 + 1.0) * (y2 - y1 + 1.0)
            parea = (px2 - px1 + 1.0) * (py2 - py1 + 1.0)
            iou = inter / (parea + areas - inter)
            mask = iou >= NMS_THRESH

            gi = (cls.astype(jnp.float32) * 0.125).astype(jnp.int32)
            si = cls - 8 * gi
            tile = pgrid_ref[i, gi]
            pgrid_ref[i, gi] = jnp.where((sub8 == si) & mask, 0.0, tile)

            onb = lane1 == box_v.reshape(1, 1)
            labels = jnp.where(onb, cls, labels)
            dead = jnp.where(lane3 == box_v, jnp.int32(1), dead)
            out.append((dead, labels))
        return tuple(out)

    carry0 = tuple((jnp.zeros((1, 1, N_PER), jnp.int32),
                    jnp.zeros((1, N_PER), jnp.int32)) for _ in range(N_IMG))
    carry = lax.fori_loop(0, N_PER, body, carry0)

    for i in range(N_IMG):
        labels = carry[i][1]
        labels_ref[i] = labels
        onehotT = (row_iota == labels).astype(jnp.float32)
        embedT_ref[i] = jnp.dot(tableT_ref[...], onehotT,
                                preferred_element_type=jnp.float32,
                                precision=lax.Precision.HIGHEST)


def kernel(sem_feats, boxes_per_cls, obj_labels, W_out, b_out, obj_embed_weight):
    del obj_labels
    n_total = N_IMG * N_PER
    featsT = sem_feats.reshape(N_IMG, N_PER, HIDDEN).transpose(0, 2, 1)
    wT = jnp.pad(W_out.T, ((0, C_PAD - NUM_CLS), (0, 0)))
    b_col = jnp.pad(b_out, (0, C_PAD - NUM_CLS)).reshape(C_PAD, 1)
    geom = boxes_per_cls.reshape(N_IMG, N_PER, NUM_CLS, 4).transpose(0, 2, 3, 1)
    geom = jnp.pad(geom, ((0, 0), (0, C_PAD - NUM_CLS), (0, 0), (0, 0)))
    tableT = jnp.pad(obj_embed_weight.T, ((0, 0), (0, C_PAD - NUM_CLS)))

    distsT, labels, embedT = pl.pallas_call(
        _decode_kernel,
        out_shape=[
            jax.ShapeDtypeStruct((N_IMG, C_PAD, N_PER), jnp.float32),
            jax.ShapeDtypeStruct((N_IMG, 1, N_PER), jnp.int32),
            jax.ShapeDtypeStruct((N_IMG, EMBED_DIM, N_PER), jnp.float32),
        ],
        scratch_shapes=[pltpu.VMEM((N_IMG, G_ROWS, 8, N_PER), jnp.float32)],
    )(featsT, wT, b_col, geom, tableT)

    obj_dists = distsT[:, :NUM_CLS, :].transpose(0, 2, 1).reshape(n_total, NUM_CLS)
    obj_preds = labels.reshape(n_total)
    obj_embed_out = embedT.transpose(0, 2, 1).reshape(n_total, EMBED_DIM)
    return (obj_dists, obj_preds, obj_embed_out)


# speedup vs baseline: 2.4807x; 2.4807x over previous
import jax
import jax.numpy as jnp
from jax import lax
from jax.experimental import pallas as pl
from jax.experimental.pallas import tpu as pltpu

NUM_CLS = 151
C_PAD = 152
G_ROWS = 19
EMBED_DIM = 200
HIDDEN = 512
N_PER = 256
N_IMG = 4
NMS_THRESH = 0.5


def _decode_kernel(featsT_ref, wT_ref, b_ref, geom_ref, tableT_ref,
                   distsT_ref, labels_ref, embedT_ref, pgrid_ref):
    row_iota = lax.broadcasted_iota(jnp.int32, (C_PAD, N_PER), 0)
    lane1 = lax.broadcasted_iota(jnp.int32, (1, N_PER), 1)
    sub8 = lax.broadcasted_iota(jnp.int32, (8, N_PER), 0)
    g3 = lax.broadcasted_iota(jnp.int32, (G_ROWS, 8, N_PER), 0)
    s3 = lax.broadcasted_iota(jnp.int32, (G_ROWS, 8, N_PER), 1)
    l3 = lax.broadcasted_iota(jnp.int32, (G_ROWS, 8, N_PER), 2)
    packed = l3 * C_PAD + g3 * 8 + s3
    big = jnp.int32(1 << 30)

    for i in range(N_IMG):
        dT = jnp.dot(wT_ref[...], featsT_ref[i],
                     preferred_element_type=jnp.float32) + b_ref[...]
        dT = jnp.where(row_iota >= NUM_CLS, -1e30, dT)
        distsT_ref[i] = dT
        m = jnp.max(dT, axis=0, keepdims=True)
        e = jnp.exp(dT - m)
        p = e / jnp.sum(e, axis=0, keepdims=True)
        p = jnp.where(row_iota == 0, -1.0, p)
        p = jnp.where(row_iota >= NUM_CLS, -1e9, p)
        pgrid_ref[i] = p.reshape(G_ROWS, 8, N_PER)

    def body(_, carry):
        out = []
        for i in range(N_IMG):
            dead, labels = carry[i]
            pall = pgrid_ref[i]
            peff = jnp.where(dead != 0, -1.0, pall)
            mx = jnp.max(peff)
            pk = jnp.min(jnp.where(peff == mx, packed, big))
            box = ((pk.astype(jnp.float32) + 0.5)
                   * (1.0 / C_PAD)).astype(jnp.int32)
            cls = pk - box * C_PAD

            g = geom_ref[i, cls]
            x1 = g[0:1]
            y1 = g[1:2]
            x2 = g[2:3]
            y2 = g[3:4]
            onb = lane1 == box
            px1 = jnp.sum(jnp.where(onb, x1, 0.0), axis=1, keepdims=True)
            py1 = jnp.sum(jnp.where(onb, y1, 0.0), axis=1, keepdims=True)
            px2 = jnp.sum(jnp.where(onb, x2, 0.0), axis=1, keepdims=True)
            py2 = jnp.sum(jnp.where(onb, y2, 0.0), axis=1, keepdims=True)

            iw = jnp.maximum(jnp.minimum(px2, x2) - jnp.maximum(px1, x1) + 1.0, 0.0)
            ih = jnp.maximum(jnp.minimum(py2, y2) - jnp.maximum(py1, y1) + 1.0, 0.0)
            inter = iw * ih
            areas = (x2 - x1 + 1.0) * (y2 - y1 + 1.0)
            parea = (px2 - px1 + 1.0) * (py2 - py1 + 1.0)
            iou = inter / (parea + areas - inter)
            mask = iou >= NMS_THRESH

            gi = (cls.astype(jnp.float32) * 0.125).astype(jnp.int32)
            si = cls - 8 * gi
            tile = pgrid_ref[i, gi]
            pgrid_ref[i, gi] = jnp.where((sub8 == si) & mask, 0.0, tile)

            labels = jnp.where(onb, cls, labels)
            dead = jnp.where(onb.reshape(1, 1, N_PER), jnp.int32(1), dead)
            out.append((dead, labels))
        return tuple(out)

    carry0 = tuple((jnp.zeros((1, 1, N_PER), jnp.int32),
                    jnp.zeros((1, N_PER), jnp.int32)) for _ in range(N_IMG))
    carry = lax.fori_loop(0, N_PER, body, carry0)

    for i in range(N_IMG):
        labels = carry[i][1]
        labels_ref[i] = labels
        onehotT = (row_iota == labels).astype(jnp.float32)
        embedT_ref[i] = jnp.dot(tableT_ref[...], onehotT,
                                preferred_element_type=jnp.float32,
                                precision=lax.Precision.HIGHEST)


def kernel(sem_feats, boxes_per_cls, obj_labels, W_out, b_out, obj_embed_weight):
    del obj_labels
    n_total = N_IMG * N_PER
    featsT = sem_feats.reshape(N_IMG, N_PER, HIDDEN).transpose(0, 2, 1)
    wT = jnp.pad(W_out.T, ((0, C_PAD - NUM_CLS), (0, 0)))
    b_col = jnp.pad(b_out, (0, C_PAD - NUM_CLS)).reshape(C_PAD, 1)
    geom = boxes_per_cls.reshape(N_IMG, N_PER, NUM_CLS, 4).transpose(0, 2, 3, 1)
    geom = jnp.pad(geom, ((0, 0), (0, C_PAD - NUM_CLS), (0, 0), (0, 0)))
    tableT = jnp.pad(obj_embed_weight.T, ((0, 0), (0, C_PAD - NUM_CLS)))

    distsT, labels, embedT = pl.pallas_call(
        _decode_kernel,
        out_shape=[
            jax.ShapeDtypeStruct((N_IMG, C_PAD, N_PER), jnp.float32),
            jax.ShapeDtypeStruct((N_IMG, 1, N_PER), jnp.int32),
            jax.ShapeDtypeStruct((N_IMG, EMBED_DIM, N_PER), jnp.float32),
        ],
        scratch_shapes=[pltpu.VMEM((N_IMG, G_ROWS, 8, N_PER), jnp.float32)],
    )(featsT, wT, b_col, geom, tableT)

    obj_dists = distsT[:, :NUM_CLS, :].transpose(0, 2, 1).reshape(n_total, NUM_CLS)
    obj_preds = labels.reshape(n_total)
    obj_embed_out = embedT.transpose(0, 2, 1).reshape(n_total, EMBED_DIM)
    return (obj_dists, obj_preds, obj_embed_out)


# speedup vs baseline: 6.4744x; 2.6099x over previous
import functools

import jax
import jax.numpy as jnp
from jax import lax
from jax.experimental import pallas as pl
from jax.experimental.pallas import tpu as pltpu
from jax.experimental.pallas import tpu_sc as plsc

NUM_CLS = 151
C_PAD = 152
EMBED_DIM = 200
E_PAD = 256
HIDDEN = 512
N_PER = 256
N_IMG = 4
NMS_THRESH = 0.5

SC_CORES = 2
SC_SUBCORES = 16
SC_WORKERS = SC_CORES * SC_SUBCORES
B_PER_W = (N_IMG * N_PER) // SC_WORKERS


def _embed_gather_sc(table_hbm, idx_hbm, out_hbm, idx_v, rows_v, sem):
    wid = lax.axis_index("s") * SC_CORES + lax.axis_index("c")
    base = wid * B_PER_W
    pltpu.sync_copy(idx_hbm.at[pl.ds(base, B_PER_W)], idx_v)
    pltpu.async_copy(table_hbm.at[idx_v], rows_v, sem).wait()
    pltpu.sync_copy(rows_v, out_hbm.at[pl.ds(base, B_PER_W)])


def _decode_kernel(featsT_ref, wT_ref, b_ref, geom_ref,
                   distsT_ref, labels_ref):
    row_iota = lax.broadcasted_iota(jnp.int32, (C_PAD, N_PER), 0)
    lane_iota = lax.broadcasted_iota(jnp.int32, (C_PAD, N_PER), 1)
    lane1 = lax.broadcasted_iota(jnp.int32, (1, N_PER), 1)
    packed = lane_iota * C_PAD + row_iota
    big = jnp.int32(1 << 30)

    ps = []
    for i in range(N_IMG):
        dT = jnp.dot(wT_ref[...], featsT_ref[i],
                     preferred_element_type=jnp.float32) + b_ref[...]
        dT = jnp.where(row_iota >= NUM_CLS, -1e30, dT)
        distsT_ref[i] = dT
        m = jnp.max(dT, axis=0, keepdims=True)
        e = jnp.exp(dT - m)
        p = e / jnp.sum(e, axis=0, keepdims=True)
        p = jnp.where(row_iota == 0, -1.0, p)
        p = jnp.where(row_iota >= NUM_CLS, -1e9, p)
        ps.append(p)

    def body(_, carry):
        out = []
        for i in range(N_IMG):
            p, labels = carry[i]
            mx = jnp.max(jnp.max(p, axis=0, keepdims=True),
                         axis=1, keepdims=True)
            pk = jnp.min(jnp.where(p == mx, packed, big))
            box = ((pk.astype(jnp.float32) + 0.5)
                   * (1.0 / C_PAD)).astype(jnp.int32)
            cls = pk - box * C_PAD

            g = geom_ref[i, cls]
            x1 = g[0:1]
            y1 = g[1:2]
            x2 = g[2:3]
            y2 = g[3:4]
            onb = lane1 == box
            px1 = jnp.sum(jnp.where(onb, x1, 0.0), axis=1, keepdims=True)
            py1 = jnp.sum(jnp.where(onb, y1, 0.0), axis=1, keepdims=True)
            px2 = jnp.sum(jnp.where(onb, x2, 0.0), axis=1, keepdims=True)
            py2 = jnp.sum(jnp.where(onb, y2, 0.0), axis=1, keepdims=True)

            iw = jnp.maximum(jnp.minimum(px2, x2) - jnp.maximum(px1, x1) + 1.0, 0.0)
            ih = jnp.maximum(jnp.minimum(py2, y2) - jnp.maximum(py1, y1) + 1.0, 0.0)
            inter = iw * ih
            areas = (x2 - x1 + 1.0) * (y2 - y1 + 1.0)
            parea = (px2 - px1 + 1.0) * (py2 - py1 + 1.0)
            iou = inter / (parea + areas - inter)
            mask = iou >= NMS_THRESH

            p = jnp.where((row_iota == cls) & mask, 0.0, p)
            p = jnp.where(lane_iota == box, -1.0, p)
            labels = jnp.where(onb, cls, labels)
            out.append((p, labels))
        return tuple(out)

    carry0 = tuple((p, jnp.zeros((1, N_PER), jnp.int32)) for p in ps)
    carry = lax.fori_loop(0, N_PER, body, carry0)

    for i in range(N_IMG):
        labels_ref[i] = carry[i][1]


def kernel(sem_feats, boxes_per_cls, obj_labels, W_out, b_out, obj_embed_weight):
    del obj_labels
    n_total = N_IMG * N_PER
    featsT = sem_feats.reshape(N_IMG, N_PER, HIDDEN).transpose(0, 2, 1)
    wT = jnp.pad(W_out.T, ((0, C_PAD - NUM_CLS), (0, 0)))
    b_col = jnp.pad(b_out, (0, C_PAD - NUM_CLS)).reshape(C_PAD, 1)
    geom = boxes_per_cls.reshape(N_IMG, N_PER, NUM_CLS, 4).transpose(0, 2, 3, 1)
    geom = jnp.pad(geom, ((0, 0), (0, C_PAD - NUM_CLS), (0, 0), (0, 0)))

    distsT, labels = pl.pallas_call(
        _decode_kernel,
        out_shape=[
            jax.ShapeDtypeStruct((N_IMG, C_PAD, N_PER), jnp.float32),
            jax.ShapeDtypeStruct((N_IMG, 1, N_PER), jnp.int32),
        ],
    )(featsT, wT, b_col, geom)

    obj_dists = distsT[:, :NUM_CLS, :].transpose(0, 2, 1).reshape(n_total, NUM_CLS)
    obj_preds = labels.reshape(n_total)

    table_p = jnp.pad(obj_embed_weight, ((0, 0), (0, E_PAD - EMBED_DIM)))
    embed_p = functools.partial(
        pl.kernel,
        out_type=jax.ShapeDtypeStruct((n_total, E_PAD), jnp.float32),
        mesh=plsc.VectorSubcoreMesh(core_axis_name="c", subcore_axis_name="s",
                                    num_cores=SC_CORES,
                                    num_subcores=SC_SUBCORES),
        scratch_types=[
            pltpu.VMEM((B_PER_W,), jnp.int32),
            pltpu.VMEM((B_PER_W, E_PAD), jnp.float32),
            pltpu.SemaphoreType.DMA,
        ],
    )(_embed_gather_sc)(table_p, obj_preds)
    obj_embed_out = embed_p[:, :EMBED_DIM]
    return (obj_dists, obj_preds, obj_embed_out)
